# baseline (speedup 1.0000x reference)
import functools

import jax
import jax.numpy as jnp
from jax import lax
from jax.experimental import pallas as pl
from jax.experimental.pallas import tpu as pltpu
from jax.experimental.pallas import tpu_sc as plsc

_NC = 2
_NS = 16
_NW = _NC * _NS
_LANES = 16

_B, _L, _F, _V, _D = 4096, 20, 26, 100000, 32
_N = _B * _L * _F
_PER_W = _N // _NW
_IW = 128
_CHUNK = 13 * _IW
_NCHUNK = _PER_W // _CHUNK
_XROWS = _CHUNK // _IW


def _tc_detranspose(tt):
    w = 4096
    grid = (_F, (_V + w - 1) // w)

    def body(in_ref, out_ref, scr):
        blk = in_ref[0]
        r = jax.lax.broadcasted_iota(jnp.int32, (_D, _D), 0)
        eye = (r == r.T).astype(jnp.float32)
        t = jax.lax.dot_general(
            blk, eye, dimension_numbers=(((0,), (0,)), ((), ())),
            precision=jax.lax.Precision.HIGHEST)
        scr[...] = t.reshape(w // 4, 4, _D)
        for q in range(4):
            out_ref[0, :, q * _D:(q + 1) * _D] = scr[:, q, :]

    return pl.pallas_call(
        body,
        grid=grid,
        in_specs=[pl.BlockSpec((1, _D, w), lambda f, g: (f, 0, g))],
        out_specs=pl.BlockSpec((1, w // 4, 128), lambda f, g: (f, g, 0)),
        out_shape=jax.ShapeDtypeStruct((_F, _V // 4, 128), jnp.float32),
        scratch_shapes=[pltpu.VMEM((w // 4, 4, _D), jnp.float32)],
    )(tt)


def _sc_gather(x1d, off1d, tbl):
    mesh = plsc.VectorSubcoreMesh(
        core_axis_name="c", subcore_axis_name="s",
        num_cores=_NC, num_subcores=_NS)

    @functools.partial(
        pl.kernel,
        out_type=jax.ShapeDtypeStruct((_N, _D), jnp.float32),
        mesh=mesh,
        scratch_types=[
            pltpu.VMEM((2, _CHUNK), jnp.int32),
            pltpu.VMEM((_CHUNK,), jnp.int32),
            pltpu.VMEM((2, _CHUNK, _D), jnp.float32),
            pltpu.SemaphoreType.DMA,
            pltpu.SemaphoreType.DMA,
            pltpu.SemaphoreType.DMA,
            pltpu.SemaphoreType.DMA,
            pltpu.SemaphoreType.DMA,
        ],
        compiler_params=pltpu.CompilerParams(use_tc_tiling_on_sc=False),
    )
    def k(x_hbm, off_hbm, tbl_hbm, out_hbm, idx_v, off_v, rows_v,
          semi0, semi1, semg, sems0, sems1):
        wid = lax.axis_index("s") * _NC + lax.axis_index("c")
        base_w = wid * _PER_W
        semi = (semi0, semi1)
        sems = (sems0, sems1)
        pltpu.sync_copy(off_hbm, off_v)
        for b in range(2):
            pltpu.async_copy(
                x_hbm.at[pl.ds(base_w + b * _CHUNK, _CHUNK)],
                idx_v.at[b], semi[b])

        @pl.loop(0, _NCHUNK, step=2)
        def _pipe(g):
            for b in range(2):
                c = g + b
                base = base_w + c * _CHUNK
                pltpu.make_async_copy(
                    x_hbm.at[pl.ds(base, _CHUNK)], idx_v.at[b],
                    semi[b]).wait()
                for t in range(_CHUNK // _LANES):
                    sl = pl.ds(t * _LANES, _LANES)
                    idx_v[b, sl] = idx_v[b, sl] + off_v[sl]
                @pl.when(c >= 2)
                def _():
                    pltpu.make_async_copy(
                        rows_v.at[b],
                        out_hbm.at[pl.ds(base - 2 * _CHUNK, _CHUNK)],
                        sems[b]).wait()
                cps = [
                    pltpu.async_copy(
                        tbl_hbm.at[idx_v.at[b, pl.ds(j * _IW, _IW)]],
                        rows_v.at[b, pl.ds(j * _IW, _IW)],
                        semg)
                    for j in range(_XROWS)
                ]
                for cp in cps:
                    cp.wait()
                pltpu.async_copy(
                    rows_v.at[b], out_hbm.at[pl.ds(base, _CHUNK)], sems[b])

                @pl.when(c + 2 < _NCHUNK)
                def _():
                    pltpu.async_copy(
                        x_hbm.at[pl.ds(base + 2 * _CHUNK, _CHUNK)],
                        idx_v.at[b], semi[b])

        for b in range(2):
            base = base_w + (_NCHUNK - 2 + b) * _CHUNK
            pltpu.make_async_copy(
                rows_v.at[b], out_hbm.at[pl.ds(base, _CHUNK)],
                sems[b]).wait()

    return k(x1d, off1d, tbl)


def kernel(x, tables):
    x1d = x.astype(jnp.int32).reshape(_N)
    off1d = jnp.tile(jnp.arange(_F, dtype=jnp.int32) * _V, _CHUNK // _F)
    tt = jnp.transpose(tables, (0, 2, 1))
    tbl = _tc_detranspose(tt).reshape(_F * _V, _D)
    out = _sc_gather(x1d, off1d, tbl)
    return out.reshape(_B, _L, _F * _D)


# speedup vs baseline: 1.4228x; 1.4228x over previous
import functools

import jax
import jax.numpy as jnp
from jax import lax
from jax.experimental import pallas as pl
from jax.experimental.pallas import tpu as pltpu
from jax.experimental.pallas import tpu_sc as plsc

_NC = 2
_NS = 16
_NW = _NC * _NS
_LANES = 16

_B, _L, _F, _V, _D = 4096, 20, 26, 100000, 32
_N = _B * _L * _F
_PER_W = _N // _NW
_IW = 128
_CHUNK = 13 * _IW
_NCHUNK = _PER_W // _CHUNK
_XROWS = _CHUNK // _IW


def _tc_detranspose(tt):
    w = 4096
    grid = (_F, (_V + w - 1) // w)

    def body(in_ref, out_ref, scr):
        blk = in_ref[0]
        r = jax.lax.broadcasted_iota(jnp.int32, (_D, _D), 0)
        eye = (r == r.T).astype(jnp.float32)
        t = jax.lax.dot_general(
            blk, eye, dimension_numbers=(((0,), (0,)), ((), ())),
            precision=jax.lax.Precision.DEFAULT)
        scr[...] = t.reshape(w // 4, 4, _D)
        for q in range(4):
            out_ref[0, :, q * _D:(q + 1) * _D] = scr[:, q, :]

    return pl.pallas_call(
        body,
        grid=grid,
        in_specs=[pl.BlockSpec((1, _D, w), lambda f, g: (f, 0, g))],
        out_specs=pl.BlockSpec((1, w // 4, 128), lambda f, g: (f, g, 0)),
        out_shape=jax.ShapeDtypeStruct((_F, _V // 4, 128), jnp.float32),
        scratch_shapes=[pltpu.VMEM((w // 4, 4, _D), jnp.float32)],
    )(tt)


def _sc_gather(x1d, off1d, tbl):
    mesh = plsc.VectorSubcoreMesh(
        core_axis_name="c", subcore_axis_name="s",
        num_cores=_NC, num_subcores=_NS)

    @functools.partial(
        pl.kernel,
        out_type=jax.ShapeDtypeStruct((_N, _D), jnp.float32),
        mesh=mesh,
        scratch_types=[
            pltpu.VMEM((2, _CHUNK), jnp.int32),
            pltpu.VMEM((_CHUNK,), jnp.int32),
            pltpu.VMEM((2, _CHUNK, _D), jnp.float32),
            pltpu.SemaphoreType.DMA,
            pltpu.SemaphoreType.DMA,
            pltpu.SemaphoreType.DMA,
            pltpu.SemaphoreType.DMA,
            pltpu.SemaphoreType.DMA,
        ],
        compiler_params=pltpu.CompilerParams(use_tc_tiling_on_sc=False),
    )
    def k(x_hbm, off_hbm, tbl_hbm, out_hbm, idx_v, off_v, rows_v,
          semi0, semi1, semg, sems0, sems1):
        wid = lax.axis_index("s") * _NC + lax.axis_index("c")
        base_w = wid * _PER_W
        semi = (semi0, semi1)
        sems = (sems0, sems1)
        pltpu.sync_copy(off_hbm, off_v)
        for b in range(2):
            pltpu.async_copy(
                x_hbm.at[pl.ds(base_w + b * _CHUNK, _CHUNK)],
                idx_v.at[b], semi[b])

        @pl.loop(0, _NCHUNK, step=2)
        def _pipe(g):
            for b in range(2):
                c = g + b
                base = base_w + c * _CHUNK
                pltpu.make_async_copy(
                    x_hbm.at[pl.ds(base, _CHUNK)], idx_v.at[b],
                    semi[b]).wait()
                for t in range(_CHUNK // _LANES):
                    sl = pl.ds(t * _LANES, _LANES)
                    idx_v[b, sl] = idx_v[b, sl] + off_v[sl]
                @pl.when(c >= 2)
                def _():
                    pltpu.make_async_copy(
                        rows_v.at[b],
                        out_hbm.at[pl.ds(base - 2 * _CHUNK, _CHUNK)],
                        sems[b]).wait()
                cps = [
                    pltpu.async_copy(
                        tbl_hbm.at[idx_v.at[b, pl.ds(j * _IW, _IW)]],
                        rows_v.at[b, pl.ds(j * _IW, _IW)],
                        semg)
                    for j in range(_XROWS)
                ]
                for cp in cps:
                    cp.wait()
                pltpu.async_copy(
                    rows_v.at[b], out_hbm.at[pl.ds(base, _CHUNK)], sems[b])

                @pl.when(c + 2 < _NCHUNK)
                def _():
                    pltpu.async_copy(
                        x_hbm.at[pl.ds(base + 2 * _CHUNK, _CHUNK)],
                        idx_v.at[b], semi[b])

        for b in range(2):
            base = base_w + (_NCHUNK - 2 + b) * _CHUNK
            pltpu.make_async_copy(
                rows_v.at[b], out_hbm.at[pl.ds(base, _CHUNK)],
                sems[b]).wait()

    return k(x1d, off1d, tbl)


def kernel(x, tables):
    x1d = x.astype(jnp.int32).reshape(_N)
    off1d = jnp.tile(jnp.arange(_F, dtype=jnp.int32) * _V, _CHUNK // _F)
    tt = jnp.transpose(tables, (0, 2, 1))
    tbl = _tc_detranspose(tt).reshape(_F * _V, _D)
    out = _sc_gather(x1d, off1d, tbl)
    return out.reshape(_B, _L, _F * _D)


# speedup vs baseline: 1.5263x; 1.0727x over previous
import functools

import jax
import jax.numpy as jnp
from jax import lax
from jax.experimental import pallas as pl
from jax.experimental.pallas import tpu as pltpu
from jax.experimental.pallas import tpu_sc as plsc

_NC = 2
_NS = 16
_NW = _NC * _NS
_LANES = 16

_B, _L, _F, _V, _D = 4096, 20, 26, 100000, 32
_N = _B * _L * _F
_PER_W = _N // _NW
_IW = 128
_CHUNK = 13 * _IW
_NCHUNK = _PER_W // _CHUNK
_XROWS = _CHUNK // _IW


def _tc_detranspose(tt):
    w = 4096
    grid = (_F, (_V + w - 1) // w)

    def body(in_ref, out_ref, scr):
        blk = in_ref[0]
        scr[...] = blk.T.reshape(w // 4, 4, _D)
        for q in range(4):
            out_ref[0, :, q * _D:(q + 1) * _D] = scr[:, q, :]

    return pl.pallas_call(
        body,
        grid=grid,
        in_specs=[pl.BlockSpec((1, _D, w), lambda f, g: (f, 0, g))],
        out_specs=pl.BlockSpec((1, w // 4, 128), lambda f, g: (f, g, 0)),
        out_shape=jax.ShapeDtypeStruct((_F, _V // 4, 128), jnp.float32),
        scratch_shapes=[pltpu.VMEM((w // 4, 4, _D), jnp.float32)],
    )(tt)


def _sc_gather(x1d, off1d, tbl):
    mesh = plsc.VectorSubcoreMesh(
        core_axis_name="c", subcore_axis_name="s",
        num_cores=_NC, num_subcores=_NS)

    @functools.partial(
        pl.kernel,
        out_type=jax.ShapeDtypeStruct((_N, _D), jnp.float32),
        mesh=mesh,
        scratch_types=[
            pltpu.VMEM((2, _CHUNK), jnp.int32),
            pltpu.VMEM((_CHUNK,), jnp.int32),
            pltpu.VMEM((2, _CHUNK, _D), jnp.float32),
            pltpu.SemaphoreType.DMA,
            pltpu.SemaphoreType.DMA,
            pltpu.SemaphoreType.DMA,
            pltpu.SemaphoreType.DMA,
            pltpu.SemaphoreType.DMA,
        ],
        compiler_params=pltpu.CompilerParams(use_tc_tiling_on_sc=False),
    )
    def k(x_hbm, off_hbm, tbl_hbm, out_hbm, idx_v, off_v, rows_v,
          semi0, semi1, semg, sems0, sems1):
        wid = lax.axis_index("s") * _NC + lax.axis_index("c")
        base_w = wid * _PER_W
        semi = (semi0, semi1)
        sems = (sems0, sems1)
        pltpu.sync_copy(off_hbm, off_v)
        for b in range(2):
            pltpu.async_copy(
                x_hbm.at[pl.ds(base_w + b * _CHUNK, _CHUNK)],
                idx_v.at[b], semi[b])

        @pl.loop(0, _NCHUNK, step=2)
        def _pipe(g):
            for b in range(2):
                c = g + b
                base = base_w + c * _CHUNK
                pltpu.make_async_copy(
                    x_hbm.at[pl.ds(base, _CHUNK)], idx_v.at[b],
                    semi[b]).wait()
                for t in range(_CHUNK // _LANES):
                    sl = pl.ds(t * _LANES, _LANES)
                    idx_v[b, sl] = idx_v[b, sl] + off_v[sl]
                @pl.when(c >= 2)
                def _():
                    pltpu.make_async_copy(
                        rows_v.at[b],
                        out_hbm.at[pl.ds(base - 2 * _CHUNK, _CHUNK)],
                        sems[b]).wait()
                cps = [
                    pltpu.async_copy(
                        tbl_hbm.at[idx_v.at[b, pl.ds(j * _IW, _IW)]],
                        rows_v.at[b, pl.ds(j * _IW, _IW)],
                        semg)
                    for j in range(_XROWS)
                ]
                for cp in cps:
                    cp.wait()
                pltpu.async_copy(
                    rows_v.at[b], out_hbm.at[pl.ds(base, _CHUNK)], sems[b])

                @pl.when(c + 2 < _NCHUNK)
                def _():
                    pltpu.async_copy(
                        x_hbm.at[pl.ds(base + 2 * _CHUNK, _CHUNK)],
                        idx_v.at[b], semi[b])

        for b in range(2):
            base = base_w + (_NCHUNK - 2 + b) * _CHUNK
            pltpu.make_async_copy(
                rows_v.at[b], out_hbm.at[pl.ds(base, _CHUNK)],
                sems[b]).wait()

    return k(x1d, off1d, tbl)


def kernel(x, tables):
    x1d = x.astype(jnp.int32).reshape(_N)
    off1d = jnp.tile(jnp.arange(_F, dtype=jnp.int32) * _V, _CHUNK // _F)
    tt = jnp.transpose(tables, (0, 2, 1))
    tbl = _tc_detranspose(tt).reshape(_F * _V, _D)
    out = _sc_gather(x1d, off1d, tbl)
    return out.reshape(_B, _L, _F * _D)
